# baseline (speedup 1.0000x reference)
import functools

import jax
import jax.numpy as jnp
from jax import lax
from jax.experimental import pallas as pl
from jax.experimental.pallas import tpu as pltpu
from jax.experimental.pallas import tpu_sc as plsc


def _make_sc_route(n_tok, topk, n_exp, n_workers):
    TK = n_tok * topk
    per_w = TK // n_workers
    n_sl = TK // 16
    w_sl = per_w // 16
    mesh = plsc.VectorSubcoreMesh(core_axis_name="c", subcore_axis_name="s")

    @functools.partial(
        pl.kernel,
        out_type=(
            jax.ShapeDtypeStruct((TK,), jnp.int32),
            jax.ShapeDtypeStruct((TK,), jnp.int32),
            jax.ShapeDtypeStruct((TK,), jnp.float32),
            jax.ShapeDtypeStruct((16,), jnp.int32),
        ),
        mesh=mesh,
        scratch_types=[
            pltpu.VMEM((TK,), jnp.int32),
            pltpu.VMEM((per_w,), jnp.int32),
            pltpu.VMEM((per_w,), jnp.int32),
            pltpu.VMEM((per_w,), jnp.float32),
            pltpu.VMEM((16,), jnp.int32),
            pltpu.SMEM((16,), jnp.int32),
            pltpu.SMEM((16,), jnp.int32),
            pltpu.SMEM((16,), jnp.int32),
            pltpu.SemaphoreType.DMA,
            pltpu.SemaphoreType.DMA,
        ],
    )
    def route(e_hbm, w_hbm, pt_hbm, pos_hbm, st_hbm, ws_hbm, off_hbm,
              e_v, pos_v, tok_v, wv_v, off_v, hist_s, pp_s, base_s,
              sem0, sem1):
        nc = lax.axis_size("c")
        wid = lax.axis_index("s") * nc + lax.axis_index("c")
        my_start = wid * per_w
        my_slice = wid * w_sl
        pltpu.sync_copy(e_hbm, e_v)
        lanes = jax.lax.iota(jnp.int32, 16)
        zero = jnp.zeros((16,), jnp.int32)
        for e in range(n_exp):
            hist_s[e] = jnp.int32(0)
            pp_s[e] = jnp.int32(0)

        def scan_body(j, c):
            @pl.when(j == my_slice)
            def _snap():
                for e in range(n_exp):
                    pp_s[e] = hist_s[e]

            v = e_v[pl.ds(j * 16, 16)]
            for l in range(16):
                ev = v[l]
                hist_s[ev] = hist_s[ev] + 1
            return c

        lax.fori_loop(0, n_sl, scan_body, jnp.int32(0))

        acc = jnp.int32(0)
        off_vec = zero
        for e in range(n_exp):
            off_vec = jnp.where(lanes == e, acc, off_vec)
            base_s[e] = acc + pp_s[e]
            acc = acc + hist_s[e]
        off_vec = jnp.where(lanes >= n_exp, acc, off_vec)

        for s in range(w_sl):
            v = e_v[pl.ds((my_slice + s) * 16, 16)]
            pos_vec = zero
            for l in range(16):
                ev = v[l]
                p = base_s[ev]
                base_s[ev] = p + 1
                pos_vec = jnp.where(lanes == l, p, pos_vec)
            pos_v[pl.ds(s * 16, 16)] = pos_vec

        pltpu.sync_copy(pos_v, pos_hbm.at[pl.ds(my_start, per_w)])
        pltpu.sync_copy(w_hbm.at[pl.ds(my_start, per_w)], wv_v)
        pltpu.sync_copy(pt_hbm.at[pl.ds(my_start, per_w)], tok_v)
        pltpu.async_copy(tok_v, st_hbm.at[pos_v], sem0).wait()
        pltpu.async_copy(wv_v, ws_hbm.at[pos_v], sem1).wait()

        @pl.when(wid == 0)
        def _offsets():
            off_v[...] = off_vec
            pltpu.sync_copy(off_v, off_hbm)

    return route


def _make_sc_gather(n_rows, dim, n_workers, chunk):
    mesh = plsc.VectorSubcoreMesh(core_axis_name="c", subcore_axis_name="s")
    per_w = n_rows // n_workers
    n_chunks = per_w // chunk

    @functools.partial(
        pl.kernel,
        out_type=jax.ShapeDtypeStruct((n_rows, dim), jnp.float32),
        mesh=mesh,
        scratch_types=[
            pltpu.VMEM((per_w,), jnp.int32),
            pltpu.VMEM((chunk, dim), jnp.float32),
            pltpu.VMEM((chunk, dim), jnp.float32),
            pltpu.SemaphoreType.DMA,
            pltpu.SemaphoreType.DMA,
        ],
    )
    def gather(x_hbm, ids_hbm, xs_hbm, idx_v, rows0, rows1, sem0, sem1):
        nc = lax.axis_size("c")
        wid = lax.axis_index("s") * nc + lax.axis_index("c")
        base_w = wid * per_w
        pltpu.sync_copy(ids_hbm.at[pl.ds(base_w, per_w)], idx_v)
        bufs = (rows0, rows1)
        sems = (sem0, sem1)
        copies = [None] * n_chunks
        for ci in range(n_chunks):
            copies[ci] = pltpu.async_copy(
                x_hbm.at[idx_v.at[pl.ds(ci * chunk, chunk)]],
                bufs[ci % 2], sems[ci % 2])
            if ci > 0:
                copies[ci - 1].wait()
                pltpu.sync_copy(bufs[(ci - 1) % 2],
                                xs_hbm.at[pl.ds(base_w + (ci - 1) * chunk,
                                                chunk)])
        copies[n_chunks - 1].wait()
        pltpu.sync_copy(bufs[(n_chunks - 1) % 2],
                        xs_hbm.at[pl.ds(base_w + (n_chunks - 1) * chunk,
                                        chunk)])

    return gather


def _make_sc_combine(n_tok, dim, topk, n_workers, chunk):
    mesh = plsc.VectorSubcoreMesh(core_axis_name="c", subcore_axis_name="s")
    per_w = n_tok // n_workers
    n_chunks = per_w // chunk
    npair = chunk * topk

    @functools.partial(
        pl.kernel,
        out_type=jax.ShapeDtypeStruct((n_tok, dim), jnp.float32),
        mesh=mesh,
        scratch_types=[
            pltpu.VMEM((per_w * topk,), jnp.int32),
            pltpu.VMEM((npair, dim), jnp.float32),
            pltpu.VMEM((npair, dim), jnp.float32),
            pltpu.VMEM((chunk, dim), jnp.float32),
            pltpu.SemaphoreType.DMA,
            pltpu.SemaphoreType.DMA,
        ],
    )
    def combine(ys_hbm, pos_hbm, out_hbm, idx_v, rows0, rows1, acc_v,
                sem0, sem1):
        nc = lax.axis_size("c")
        wid = lax.axis_index("s") * nc + lax.axis_index("c")
        base_t = wid * per_w
        pltpu.sync_copy(pos_hbm.at[pl.ds(base_t * topk, per_w * topk)], idx_v)
        bufs = (rows0, rows1)
        sems = (sem0, sem1)
        copies = [None] * n_chunks

        def compute(rows_v, ci):
            def body(j, _):
                off = j * 16
                for t in range(chunk):
                    v = rows_v[t * topk, pl.ds(off, 16)]
                    for k in range(1, topk):
                        v = v + rows_v[t * topk + k, pl.ds(off, 16)]
                    acc_v[t, pl.ds(off, 16)] = v
                return 0

            lax.fori_loop(0, dim // 16, body, 0)
            pltpu.sync_copy(acc_v,
                            out_hbm.at[pl.ds(base_t + ci * chunk, chunk)])

        for ci in range(n_chunks):
            copies[ci] = pltpu.async_copy(
                ys_hbm.at[idx_v.at[pl.ds(ci * npair, npair)]],
                bufs[ci % 2], sems[ci % 2])
            if ci > 0:
                copies[ci - 1].wait()
                compute(bufs[(ci - 1) % 2], ci - 1)
        copies[n_chunks - 1].wait()
        compute(bufs[(n_chunks - 1) % 2], n_chunks - 1)

    return combine


def _grouped_ffn_body(bids_ref, eids_ref, valids_ref, offs_ref,
                      xs_ref, g_ref, u_ref, d_ref, ws_ref, out_ref,
                      *, blk, inter):
    g = pl.program_id(0)
    b = bids_ref[g]
    e = eids_ref[g]
    valid = valids_ref[g]
    row0 = b * blk
    lo = jnp.clip(offs_ref[e] - row0, 0, blk)
    hi = jnp.clip(offs_ref[e + 1] - row0, 0, blk)
    hi = jnp.where(valid > 0, hi, lo)

    xb = xs_ref[...]
    gg = jax.lax.dot_general(xb, g_ref[...], (((1,), (1,)), ((), ())),
                             preferred_element_type=jnp.float32)
    uu = jax.lax.dot_general(xb, u_ref[...], (((1,), (1,)), ((), ())),
                             preferred_element_type=jnp.float32)
    h = gg * jax.nn.sigmoid(gg) * uu

    rows = jax.lax.broadcasted_iota(jnp.int32, (blk, 1), 0)
    inrange = (rows >= lo) & (rows < hi)
    ws = ws_ref[...]
    h = h * jnp.where(inrange, ws, 0.0)

    y = jax.lax.dot_general(h, d_ref[...],
                            (((1,), (1,)), ((), ())),
                            preferred_element_type=jnp.float32)

    prev = bids_ref[jnp.maximum(g - 1, 0)]
    is_first = (g == 0) | (prev != b)

    @pl.when(is_first)
    def _init():
        out_ref[...] = y

    @pl.when(jnp.logical_not(is_first))
    def _acc():
        out_ref[...] += y


def kernel(x, token_to_expert_indices, weights, gate_proj_weight,
           up_proj_weight, down_proj_weight):
    T, DIM = x.shape
    E, INTER, _ = gate_proj_weight.shape
    TOPK = token_to_expert_indices.shape[1]
    TK = T * TOPK
    BLK = 128
    NB = TK // BLK
    G = NB + E - 1

    e_flat = token_to_expert_indices.reshape(-1).astype(jnp.int32)
    w_flat = weights.reshape(-1)
    iota = jnp.arange(TK, dtype=jnp.int32)
    perm = jnp.argsort(e_flat, stable=True).astype(jnp.int32)
    e_sorted = e_flat[perm]
    pos = jnp.zeros((TK,), jnp.int32).at[perm].set(iota)
    token_ids = perm // TOPK
    w_sorted = w_flat[perm]
    offsets = jnp.searchsorted(
        e_sorted, jnp.arange(E + 1, dtype=jnp.int32)).astype(jnp.int32)
    counts = offsets[1:] - offsets[:E]

    b0 = offsets[:E] // BLK
    b1 = jnp.maximum(offsets[1:] - 1, 0) // BLK
    nb_e = jnp.where(counts > 0, b1 - b0 + 1, 0)
    cum = jnp.cumsum(nb_e)
    gs = jnp.arange(G, dtype=jnp.int32)
    eids = jnp.searchsorted(cum, gs, side="right").astype(jnp.int32)
    valids = (gs < cum[E - 1]).astype(jnp.int32)
    eids = jnp.clip(eids, 0, E - 1)
    start = jnp.concatenate([jnp.zeros((1,), jnp.int32),
                             cum.astype(jnp.int32)])[eids]
    bids = b0[eids] + (gs - start)
    bids = jnp.where(valids > 0, bids, NB - 1).astype(jnp.int32)

    gather = _make_sc_gather(TK, DIM, 32, 16)
    xs = gather(x, token_ids)

    w2 = w_sorted.reshape(TK, 1)

    grid_spec = pltpu.PrefetchScalarGridSpec(
        num_scalar_prefetch=4,
        grid=(G,),
        in_specs=[
            pl.BlockSpec((BLK, DIM), lambda g, bids, eids, valids, offs: (bids[g], 0)),
            pl.BlockSpec((None, INTER, DIM), lambda g, bids, eids, valids, offs: (eids[g], 0, 0)),
            pl.BlockSpec((None, INTER, DIM), lambda g, bids, eids, valids, offs: (eids[g], 0, 0)),
            pl.BlockSpec((None, DIM, INTER), lambda g, bids, eids, valids, offs: (eids[g], 0, 0)),
            pl.BlockSpec((BLK, 1), lambda g, bids, eids, valids, offs: (bids[g], 0)),
        ],
        out_specs=pl.BlockSpec((BLK, DIM), lambda g, bids, eids, valids, offs: (bids[g], 0)),
    )
    ys = pl.pallas_call(
        functools.partial(_grouped_ffn_body, blk=BLK, inter=INTER),
        grid_spec=grid_spec,
        out_shape=jax.ShapeDtypeStruct((TK, DIM), jnp.float32),
        compiler_params=pltpu.CompilerParams(
            dimension_semantics=("arbitrary",),
        ),
    )(bids, eids, valids, offsets, xs, gate_proj_weight, up_proj_weight,
      down_proj_weight, w2)

    combine = _make_sc_combine(T, DIM, TOPK, 32, 8)
    out = combine(ys, pos)
    return out


# speedup vs baseline: 1.1291x; 1.1291x over previous
import functools

import jax
import jax.numpy as jnp
from jax import lax
from jax.experimental import pallas as pl
from jax.experimental.pallas import tpu as pltpu
from jax.experimental.pallas import tpu_sc as plsc


def _make_sc_route(n_tok, topk, n_exp, n_workers):
    TK = n_tok * topk
    per_w = TK // n_workers
    n_sl = TK // 16
    w_sl = per_w // 16
    mesh = plsc.VectorSubcoreMesh(core_axis_name="c", subcore_axis_name="s")

    @functools.partial(
        pl.kernel,
        out_type=(
            jax.ShapeDtypeStruct((TK,), jnp.int32),
            jax.ShapeDtypeStruct((TK,), jnp.int32),
            jax.ShapeDtypeStruct((TK,), jnp.float32),
            jax.ShapeDtypeStruct((16,), jnp.int32),
        ),
        mesh=mesh,
        scratch_types=[
            pltpu.VMEM((TK,), jnp.int32),
            pltpu.VMEM((per_w,), jnp.int32),
            pltpu.VMEM((per_w,), jnp.int32),
            pltpu.VMEM((per_w,), jnp.float32),
            pltpu.VMEM((16,), jnp.int32),
            pltpu.SMEM((16,), jnp.int32),
            pltpu.SMEM((16,), jnp.int32),
            pltpu.SMEM((16,), jnp.int32),
            pltpu.SemaphoreType.DMA,
            pltpu.SemaphoreType.DMA,
        ],
    )
    def route(e_hbm, w_hbm, pt_hbm, pos_hbm, st_hbm, ws_hbm, off_hbm,
              e_v, pos_v, tok_v, wv_v, off_v, hist_s, pp_s, base_s,
              sem0, sem1):
        nc = lax.axis_size("c")
        wid = lax.axis_index("s") * nc + lax.axis_index("c")
        my_start = wid * per_w
        my_slice = wid * w_sl
        pltpu.sync_copy(e_hbm, e_v)
        lanes = jax.lax.iota(jnp.int32, 16)
        zero = jnp.zeros((16,), jnp.int32)
        for e in range(n_exp):
            hist_s[e] = jnp.int32(0)
            pp_s[e] = jnp.int32(0)

        def scan_body(j, c):
            @pl.when(j == my_slice)
            def _snap():
                for e in range(n_exp):
                    pp_s[e] = hist_s[e]

            v = e_v[pl.ds(j * 16, 16)]
            for l in range(16):
                ev = v[l]
                hist_s[ev] = hist_s[ev] + 1
            return c

        lax.fori_loop(0, n_sl, scan_body, jnp.int32(0))

        acc = jnp.int32(0)
        off_vec = zero
        for e in range(n_exp):
            off_vec = jnp.where(lanes == e, acc, off_vec)
            base_s[e] = acc + pp_s[e]
            acc = acc + hist_s[e]
        off_vec = jnp.where(lanes >= n_exp, acc, off_vec)

        for s in range(w_sl):
            v = e_v[pl.ds((my_slice + s) * 16, 16)]
            pos_vec = zero
            for l in range(16):
                ev = v[l]
                p = base_s[ev]
                base_s[ev] = p + 1
                pos_vec = jnp.where(lanes == l, p, pos_vec)
            pos_v[pl.ds(s * 16, 16)] = pos_vec

        pltpu.sync_copy(pos_v, pos_hbm.at[pl.ds(my_start, per_w)])
        pltpu.sync_copy(w_hbm.at[pl.ds(my_start, per_w)], wv_v)
        pltpu.sync_copy(pt_hbm.at[pl.ds(my_start, per_w)], tok_v)
        pltpu.async_copy(tok_v, st_hbm.at[pos_v], sem0).wait()
        pltpu.async_copy(wv_v, ws_hbm.at[pos_v], sem1).wait()

        @pl.when(wid == 0)
        def _offsets():
            off_v[...] = off_vec
            pltpu.sync_copy(off_v, off_hbm)

    return route


def _make_sc_gather(n_rows, dim, n_workers, chunk):
    mesh = plsc.VectorSubcoreMesh(core_axis_name="c", subcore_axis_name="s")
    per_w = n_rows // n_workers
    n_chunks = per_w // chunk

    @functools.partial(
        pl.kernel,
        out_type=jax.ShapeDtypeStruct((n_rows, dim), jnp.float32),
        mesh=mesh,
        scratch_types=[
            pltpu.VMEM((per_w,), jnp.int32),
            pltpu.VMEM((chunk, dim), jnp.float32),
            pltpu.VMEM((chunk, dim), jnp.float32),
            pltpu.SemaphoreType.DMA,
            pltpu.SemaphoreType.DMA,
        ],
    )
    def gather(x_hbm, ids_hbm, xs_hbm, idx_v, rows0, rows1, sem0, sem1):
        nc = lax.axis_size("c")
        wid = lax.axis_index("s") * nc + lax.axis_index("c")
        base_w = wid * per_w
        pltpu.sync_copy(ids_hbm.at[pl.ds(base_w, per_w)], idx_v)
        bufs = (rows0, rows1)
        sems = (sem0, sem1)
        copies = [None] * n_chunks
        for ci in range(n_chunks):
            copies[ci] = pltpu.async_copy(
                x_hbm.at[idx_v.at[pl.ds(ci * chunk, chunk)]],
                bufs[ci % 2], sems[ci % 2])
            if ci > 0:
                copies[ci - 1].wait()
                pltpu.sync_copy(bufs[(ci - 1) % 2],
                                xs_hbm.at[pl.ds(base_w + (ci - 1) * chunk,
                                                chunk)])
        copies[n_chunks - 1].wait()
        pltpu.sync_copy(bufs[(n_chunks - 1) % 2],
                        xs_hbm.at[pl.ds(base_w + (n_chunks - 1) * chunk,
                                        chunk)])

    return gather


def _make_sc_combine(n_tok, dim, topk, n_workers, chunk):
    mesh = plsc.VectorSubcoreMesh(core_axis_name="c", subcore_axis_name="s")
    per_w = n_tok // n_workers
    n_chunks = per_w // chunk
    npair = chunk * topk

    @functools.partial(
        pl.kernel,
        out_type=jax.ShapeDtypeStruct((n_tok, dim), jnp.float32),
        mesh=mesh,
        scratch_types=[
            pltpu.VMEM((per_w * topk,), jnp.int32),
            pltpu.VMEM((npair, dim), jnp.float32),
            pltpu.VMEM((npair, dim), jnp.float32),
            pltpu.VMEM((chunk, dim), jnp.float32),
            pltpu.SemaphoreType.DMA,
            pltpu.SemaphoreType.DMA,
        ],
    )
    def combine(ys_hbm, pos_hbm, out_hbm, idx_v, rows0, rows1, acc_v,
                sem0, sem1):
        nc = lax.axis_size("c")
        wid = lax.axis_index("s") * nc + lax.axis_index("c")
        base_t = wid * per_w
        pltpu.sync_copy(pos_hbm.at[pl.ds(base_t * topk, per_w * topk)], idx_v)
        bufs = (rows0, rows1)
        sems = (sem0, sem1)
        copies = [None] * n_chunks

        def compute(rows_v, ci):
            def body(j, _):
                off = j * 16
                for t in range(chunk):
                    v = rows_v[t * topk, pl.ds(off, 16)]
                    for k in range(1, topk):
                        v = v + rows_v[t * topk + k, pl.ds(off, 16)]
                    acc_v[t, pl.ds(off, 16)] = v
                return 0

            lax.fori_loop(0, dim // 16, body, 0)
            pltpu.sync_copy(acc_v,
                            out_hbm.at[pl.ds(base_t + ci * chunk, chunk)])

        for ci in range(n_chunks):
            copies[ci] = pltpu.async_copy(
                ys_hbm.at[idx_v.at[pl.ds(ci * npair, npair)]],
                bufs[ci % 2], sems[ci % 2])
            if ci > 0:
                copies[ci - 1].wait()
                compute(bufs[(ci - 1) % 2], ci - 1)
        copies[n_chunks - 1].wait()
        compute(bufs[(n_chunks - 1) % 2], n_chunks - 1)

    return combine


def _grouped_ffn_body(bids_ref, eids_ref, valids_ref, offs_ref,
                      xs_ref, g_ref, u_ref, d_ref, ws_ref, *rest,
                      blk, inter, has_prev=False):
    out_ref = rest[-1]
    g = pl.program_id(0)
    b = bids_ref[g]
    e = eids_ref[g]
    valid = valids_ref[g]
    row0 = b * blk
    lo = jnp.clip(offs_ref[e] - row0, 0, blk)
    hi = jnp.clip(offs_ref[e + 1] - row0, 0, blk)
    hi = jnp.where(valid > 0, hi, lo)

    xb = xs_ref[...]
    gg = jax.lax.dot_general(xb, g_ref[...], (((1,), (1,)), ((), ())),
                             preferred_element_type=jnp.float32)
    uu = jax.lax.dot_general(xb, u_ref[...], (((1,), (1,)), ((), ())),
                             preferred_element_type=jnp.float32)
    h = gg * jax.nn.sigmoid(gg) * uu

    rows = jax.lax.broadcasted_iota(jnp.int32, (blk, 1), 0)
    inrange = (rows >= lo) & (rows < hi)
    ws = ws_ref[...]
    h = h * jnp.where(inrange, ws, 0.0)

    y = jax.lax.dot_general(h, d_ref[...],
                            (((1,), (1,)), ((), ())),
                            preferred_element_type=jnp.float32)

    prev = bids_ref[jnp.maximum(g - 1, 0)]
    is_first = (g == 0) | (prev != b)

    @pl.when(is_first)
    def _init():
        out_ref[...] = y

    @pl.when(jnp.logical_not(is_first))
    def _acc():
        out_ref[...] += y


def kernel(x, token_to_expert_indices, weights, gate_proj_weight,
           up_proj_weight, down_proj_weight):
    T, DIM = x.shape
    E, INTER, _ = gate_proj_weight.shape
    TOPK = token_to_expert_indices.shape[1]
    TK = T * TOPK
    BLK = 256
    NB = TK // BLK
    G = NB + E - 1

    e_flat = token_to_expert_indices.reshape(-1).astype(jnp.int32)
    w_flat = weights.reshape(-1)
    iota = jnp.arange(TK, dtype=jnp.int32)
    perm = jnp.argsort(e_flat, stable=True).astype(jnp.int32)
    e_sorted = e_flat[perm]
    pos = jnp.zeros((TK,), jnp.int32).at[perm].set(iota)
    token_ids = perm // TOPK
    w_sorted = w_flat[perm]
    offsets = jnp.searchsorted(
        e_sorted, jnp.arange(E + 1, dtype=jnp.int32)).astype(jnp.int32)
    counts = offsets[1:] - offsets[:E]

    HALF = TK // 2
    NBH = HALF // BLK
    GH = NBH + E - 1
    gather = _make_sc_gather(HALF, DIM, 32, 16)
    w2 = w_sorted.reshape(TK, 1)

    def half_meta(offs_clip, lo):
        offh = offs_clip - lo
        cnts = offh[1:] - offh[:E]
        b0 = offh[:E] // BLK
        b1 = jnp.maximum(offh[1:] - 1, 0) // BLK
        nb_e = jnp.where(cnts > 0, b1 - b0 + 1, 0)
        cum = jnp.cumsum(nb_e)
        gs = jnp.arange(GH, dtype=jnp.int32)
        eids = jnp.searchsorted(cum, gs, side="right").astype(jnp.int32)
        valids = (gs < cum[E - 1]).astype(jnp.int32)
        eids = jnp.clip(eids, 0, E - 1)
        start = jnp.concatenate([jnp.zeros((1,), jnp.int32),
                                 cum.astype(jnp.int32)])[eids]
        bids = b0[eids] + (gs - start)
        bids = jnp.where(valids > 0, bids, NBH - 1).astype(jnp.int32)
        return bids, eids, valids, offh

    ys = None
    for h in range(2):
        lo = h * HALF
        xs_h = gather(x, jax.lax.dynamic_slice(token_ids, (lo,), (HALF,)))
        bids, eids, valids, offh = half_meta(
            jnp.clip(offsets, lo, lo + HALF), lo)
        w2h = jax.lax.dynamic_slice(w2, (lo, 0), (HALF, 1))

        def mk_spec(nbase=h * NBH):
            return pl.BlockSpec(
                (BLK, DIM),
                lambda g, bids, eids, valids, offs: (nbase + bids[g], 0))

        in_specs = [
            pl.BlockSpec((BLK, DIM),
                         lambda g, bids, eids, valids, offs: (bids[g], 0)),
            pl.BlockSpec((None, INTER, DIM),
                         lambda g, bids, eids, valids, offs: (eids[g], 0, 0)),
            pl.BlockSpec((None, INTER, DIM),
                         lambda g, bids, eids, valids, offs: (eids[g], 0, 0)),
            pl.BlockSpec((None, DIM, INTER),
                         lambda g, bids, eids, valids, offs: (eids[g], 0, 0)),
            pl.BlockSpec((BLK, 1),
                         lambda g, bids, eids, valids, offs: (bids[g], 0)),
        ]
        args = [bids, eids, valids, offh, xs_h, gate_proj_weight,
                up_proj_weight, down_proj_weight, w2h]
        io_alias = {}
        if ys is not None:
            in_specs.append(pl.BlockSpec(memory_space=pl.ANY))
            args.append(ys)
            io_alias = {9: 0}
        grid_spec = pltpu.PrefetchScalarGridSpec(
            num_scalar_prefetch=4,
            grid=(GH,),
            in_specs=in_specs,
            out_specs=mk_spec(),
        )
        ys = pl.pallas_call(
            functools.partial(_grouped_ffn_body, blk=BLK, inter=INTER,
                              has_prev=ys is not None),
            grid_spec=grid_spec,
            out_shape=jax.ShapeDtypeStruct((TK, DIM), jnp.float32),
            input_output_aliases=io_alias,
            compiler_params=pltpu.CompilerParams(
                dimension_semantics=("arbitrary",),
            ),
        )(*args)

    combine = _make_sc_combine(T, DIM, TOPK, 32, 8)
    out = combine(ys, pos)
    return out


# speedup vs baseline: 1.3670x; 1.2107x over previous
import functools

import jax
import jax.numpy as jnp
from jax import lax
from jax.experimental import pallas as pl
from jax.experimental.pallas import tpu as pltpu
from jax.experimental.pallas import tpu_sc as plsc


def _make_sc_route(n_tok, topk, n_exp, n_workers):
    TK = n_tok * topk
    per_w = TK // n_workers
    n_sl = TK // 16
    w_sl = per_w // 16
    mesh = plsc.VectorSubcoreMesh(core_axis_name="c", subcore_axis_name="s")

    @functools.partial(
        pl.kernel,
        out_type=(
            jax.ShapeDtypeStruct((TK,), jnp.int32),
            jax.ShapeDtypeStruct((TK,), jnp.int32),
            jax.ShapeDtypeStruct((TK,), jnp.float32),
            jax.ShapeDtypeStruct((16,), jnp.int32),
        ),
        mesh=mesh,
        scratch_types=[
            pltpu.VMEM((TK,), jnp.int32),
            pltpu.VMEM((per_w,), jnp.int32),
            pltpu.VMEM((per_w,), jnp.int32),
            pltpu.VMEM((per_w,), jnp.float32),
            pltpu.VMEM((16,), jnp.int32),
            pltpu.SMEM((16,), jnp.int32),
            pltpu.SMEM((16,), jnp.int32),
            pltpu.SMEM((16,), jnp.int32),
            pltpu.SemaphoreType.DMA,
            pltpu.SemaphoreType.DMA,
        ],
    )
    def route(e_hbm, w_hbm, pt_hbm, pos_hbm, st_hbm, ws_hbm, off_hbm,
              e_v, pos_v, tok_v, wv_v, off_v, hist_s, pp_s, base_s,
              sem0, sem1):
        nc = lax.axis_size("c")
        wid = lax.axis_index("s") * nc + lax.axis_index("c")
        my_start = wid * per_w
        my_slice = wid * w_sl
        pltpu.sync_copy(e_hbm, e_v)
        lanes = jax.lax.iota(jnp.int32, 16)
        zero = jnp.zeros((16,), jnp.int32)
        for e in range(n_exp):
            hist_s[e] = jnp.int32(0)
            pp_s[e] = jnp.int32(0)

        def scan_body(j, c):
            @pl.when(j == my_slice)
            def _snap():
                for e in range(n_exp):
                    pp_s[e] = hist_s[e]

            v = e_v[pl.ds(j * 16, 16)]
            for l in range(16):
                ev = v[l]
                hist_s[ev] = hist_s[ev] + 1
            return c

        lax.fori_loop(0, n_sl, scan_body, jnp.int32(0))

        acc = jnp.int32(0)
        off_vec = zero
        for e in range(n_exp):
            off_vec = jnp.where(lanes == e, acc, off_vec)
            base_s[e] = acc + pp_s[e]
            acc = acc + hist_s[e]
        off_vec = jnp.where(lanes >= n_exp, acc, off_vec)

        for s in range(w_sl):
            v = e_v[pl.ds((my_slice + s) * 16, 16)]
            pos_vec = zero
            for l in range(16):
                ev = v[l]
                p = base_s[ev]
                base_s[ev] = p + 1
                pos_vec = jnp.where(lanes == l, p, pos_vec)
            pos_v[pl.ds(s * 16, 16)] = pos_vec

        pltpu.sync_copy(pos_v, pos_hbm.at[pl.ds(my_start, per_w)])
        pltpu.sync_copy(w_hbm.at[pl.ds(my_start, per_w)], wv_v)
        pltpu.sync_copy(pt_hbm.at[pl.ds(my_start, per_w)], tok_v)
        pltpu.async_copy(tok_v, st_hbm.at[pos_v], sem0).wait()
        pltpu.async_copy(wv_v, ws_hbm.at[pos_v], sem1).wait()

        @pl.when(wid == 0)
        def _offsets():
            off_v[...] = off_vec
            pltpu.sync_copy(off_v, off_hbm)

    return route


def _make_sc_gather(n_rows, dim, n_workers, chunk):
    mesh = plsc.VectorSubcoreMesh(core_axis_name="c", subcore_axis_name="s")
    per_w = n_rows // n_workers
    n_chunks = per_w // chunk

    @functools.partial(
        pl.kernel,
        out_type=jax.ShapeDtypeStruct((n_rows, dim), jnp.float32),
        mesh=mesh,
        scratch_types=[
            pltpu.VMEM((per_w,), jnp.int32),
            pltpu.VMEM((chunk, dim), jnp.float32),
            pltpu.VMEM((chunk, dim), jnp.float32),
            pltpu.SemaphoreType.DMA,
            pltpu.SemaphoreType.DMA,
        ],
    )
    def gather(x_hbm, ids_hbm, xs_hbm, idx_v, rows0, rows1, sem0, sem1):
        nc = lax.axis_size("c")
        wid = lax.axis_index("s") * nc + lax.axis_index("c")
        base_w = wid * per_w
        pltpu.sync_copy(ids_hbm.at[pl.ds(base_w, per_w)], idx_v)
        bufs = (rows0, rows1)
        sems = (sem0, sem1)
        copies = [None] * n_chunks
        for ci in range(n_chunks):
            copies[ci] = pltpu.async_copy(
                x_hbm.at[idx_v.at[pl.ds(ci * chunk, chunk)]],
                bufs[ci % 2], sems[ci % 2])
            if ci > 0:
                copies[ci - 1].wait()
                pltpu.sync_copy(bufs[(ci - 1) % 2],
                                xs_hbm.at[pl.ds(base_w + (ci - 1) * chunk,
                                                chunk)])
        copies[n_chunks - 1].wait()
        pltpu.sync_copy(bufs[(n_chunks - 1) % 2],
                        xs_hbm.at[pl.ds(base_w + (n_chunks - 1) * chunk,
                                        chunk)])

    return gather


def _make_sc_combine(n_tok, dim, topk, n_workers, chunk):
    mesh = plsc.VectorSubcoreMesh(core_axis_name="c", subcore_axis_name="s")
    per_w = n_tok // n_workers
    n_chunks = per_w // chunk
    npair = chunk * topk

    @functools.partial(
        pl.kernel,
        out_type=jax.ShapeDtypeStruct((n_tok, dim), jnp.float32),
        mesh=mesh,
        scratch_types=[
            pltpu.VMEM((per_w * topk,), jnp.int32),
            pltpu.VMEM((npair, dim), jnp.float32),
            pltpu.VMEM((npair, dim), jnp.float32),
            pltpu.VMEM((chunk, dim), jnp.float32),
            pltpu.SemaphoreType.DMA,
            pltpu.SemaphoreType.DMA,
        ],
    )
    def combine(ys_hbm, pos_hbm, out_hbm, idx_v, rows0, rows1, acc_v,
                sem0, sem1):
        nc = lax.axis_size("c")
        wid = lax.axis_index("s") * nc + lax.axis_index("c")
        base_t = wid * per_w
        pltpu.sync_copy(pos_hbm.at[pl.ds(base_t * topk, per_w * topk)], idx_v)
        bufs = (rows0, rows1)
        sems = (sem0, sem1)
        copies = [None] * n_chunks

        def compute(rows_v, ci):
            def body(j, _):
                off = j * 16
                for t in range(chunk):
                    v = rows_v[t * topk, pl.ds(off, 16)]
                    for k in range(1, topk):
                        v = v + rows_v[t * topk + k, pl.ds(off, 16)]
                    acc_v[t, pl.ds(off, 16)] = v
                return 0

            lax.fori_loop(0, dim // 16, body, 0)
            pltpu.sync_copy(acc_v,
                            out_hbm.at[pl.ds(base_t + ci * chunk, chunk)])

        for ci in range(n_chunks):
            copies[ci] = pltpu.async_copy(
                ys_hbm.at[idx_v.at[pl.ds(ci * npair, npair)]],
                bufs[ci % 2], sems[ci % 2])
            if ci > 0:
                copies[ci - 1].wait()
                compute(bufs[(ci - 1) % 2], ci - 1)
        copies[n_chunks - 1].wait()
        compute(bufs[(n_chunks - 1) % 2], n_chunks - 1)

    return combine


def _grouped_ffn_body(bids_ref, eids_ref, valids_ref, offs_ref,
                      xs_ref, g_ref, u_ref, d_ref, ws_ref, *rest,
                      blk, inter, has_prev=False):
    out_ref = rest[-1]
    g = pl.program_id(0)
    b = bids_ref[g]
    e = eids_ref[g]
    valid = valids_ref[g]
    row0 = b * blk
    lo = jnp.clip(offs_ref[e] - row0, 0, blk)
    hi = jnp.clip(offs_ref[e + 1] - row0, 0, blk)
    hi = jnp.where(valid > 0, hi, lo)

    xb = xs_ref[...]
    gg = jax.lax.dot_general(xb, g_ref[...], (((1,), (1,)), ((), ())),
                             preferred_element_type=jnp.float32)
    uu = jax.lax.dot_general(xb, u_ref[...], (((1,), (1,)), ((), ())),
                             preferred_element_type=jnp.float32)
    h = gg * jax.nn.sigmoid(gg) * uu

    rows = jax.lax.broadcasted_iota(jnp.int32, (blk, 1), 0)
    inrange = (rows >= lo) & (rows < hi)
    ws = ws_ref[...]
    h = h * jnp.where(inrange, ws, 0.0)

    y = jax.lax.dot_general(h, d_ref[...],
                            (((1,), (1,)), ((), ())),
                            preferred_element_type=jnp.float32)

    prev = bids_ref[jnp.maximum(g - 1, 0)]
    is_first = (g == 0) | (prev != b)

    @pl.when(is_first)
    def _init():
        out_ref[...] = y

    @pl.when(jnp.logical_not(is_first))
    def _acc():
        out_ref[...] += y


def kernel(x, token_to_expert_indices, weights, gate_proj_weight,
           up_proj_weight, down_proj_weight):
    T, DIM = x.shape
    E, INTER, _ = gate_proj_weight.shape
    TOPK = token_to_expert_indices.shape[1]
    TK = T * TOPK
    BLK = 256
    NB = TK // BLK
    G = NB + E - 1

    e_flat = token_to_expert_indices.reshape(-1).astype(jnp.int32)
    w_flat = weights.reshape(-1)
    iota = jnp.arange(TK, dtype=jnp.int32)
    perm = jnp.argsort(e_flat, stable=True).astype(jnp.int32)
    e_sorted = e_flat[perm]
    pos = jnp.zeros((TK,), jnp.int32).at[perm].set(iota)
    token_ids = perm // TOPK
    w_sorted = w_flat[perm]
    offsets = jnp.searchsorted(
        e_sorted, jnp.arange(E + 1, dtype=jnp.int32)).astype(jnp.int32)
    counts = offsets[1:] - offsets[:E]

    b0 = offsets[:E] // BLK
    b1 = jnp.maximum(offsets[1:] - 1, 0) // BLK
    nb_e = jnp.where(counts > 0, b1 - b0 + 1, 0)
    cum = jnp.cumsum(nb_e)
    gs = jnp.arange(G, dtype=jnp.int32)
    eids = jnp.searchsorted(cum, gs, side="right").astype(jnp.int32)
    valids = (gs < cum[E - 1]).astype(jnp.int32)
    eids = jnp.clip(eids, 0, E - 1)
    start = jnp.concatenate([jnp.zeros((1,), jnp.int32),
                             cum.astype(jnp.int32)])[eids]
    bids = b0[eids] + (gs - start)
    bids = jnp.where(valids > 0, bids, NB - 1).astype(jnp.int32)

    gather = _make_sc_gather(TK, DIM, 32, 16)
    xs = gather(x, token_ids)

    w2 = w_sorted.reshape(TK, 1)

    grid_spec = pltpu.PrefetchScalarGridSpec(
        num_scalar_prefetch=4,
        grid=(G,),
        in_specs=[
            pl.BlockSpec((BLK, DIM), lambda g, bids, eids, valids, offs: (bids[g], 0)),
            pl.BlockSpec((None, INTER, DIM), lambda g, bids, eids, valids, offs: (eids[g], 0, 0)),
            pl.BlockSpec((None, INTER, DIM), lambda g, bids, eids, valids, offs: (eids[g], 0, 0)),
            pl.BlockSpec((None, DIM, INTER), lambda g, bids, eids, valids, offs: (eids[g], 0, 0)),
            pl.BlockSpec((BLK, 1), lambda g, bids, eids, valids, offs: (bids[g], 0)),
        ],
        out_specs=pl.BlockSpec((BLK, DIM), lambda g, bids, eids, valids, offs: (bids[g], 0)),
    )
    ys = pl.pallas_call(
        functools.partial(_grouped_ffn_body, blk=BLK, inter=INTER),
        grid_spec=grid_spec,
        out_shape=jax.ShapeDtypeStruct((TK, DIM), jnp.float32),
        compiler_params=pltpu.CompilerParams(
            dimension_semantics=("arbitrary",),
        ),
    )(bids, eids, valids, offsets, xs, gate_proj_weight, up_proj_weight,
      down_proj_weight, w2)

    combine = _make_sc_combine(T, DIM, TOPK, 32, 8)
    out = combine(ys, pos)
    return out
